# Initial kernel scaffold; baseline (speedup 1.0000x reference)
import jax
import jax.numpy as jnp
from jax.experimental import pallas as pl


def kernel(hidden_states, W, b):
    raise NotImplementedError("write your pallas kernel here")


# speedup vs baseline: 4.8320x; 4.8320x over previous
import functools

import jax
import jax.numpy as jnp
from jax.experimental import pallas as pl
from jax.experimental.pallas import tpu as pltpu

TOP_K = 8
NUM_EXPERTS = 64
HIDDEN = 4096
TOKEN_BLOCK = 512


def _router_block(x_ref, wt_ref, b_ref, scores_ref, idx_ref):
    x = x_ref[...]
    logits = jnp.dot(x, wt_ref[...], preferred_element_type=jnp.float32)
    logits = logits + b_ref[...]

    tb = logits.shape[0]
    e_iota = jax.lax.broadcasted_iota(jnp.int32, (tb, NUM_EXPERTS), 1)

    vals = logits
    selected = jnp.zeros((tb, NUM_EXPERTS), dtype=jnp.bool_)
    top_max = None
    idx_cols = []
    for k in range(TOP_K):
        m = jnp.max(vals, axis=1, keepdims=True)
        hit = vals == m
        idx = jnp.min(jnp.where(hit, e_iota, NUM_EXPERTS), axis=1, keepdims=True)
        chosen = e_iota == idx
        selected = jnp.logical_or(selected, chosen)
        vals = jnp.where(chosen, -jnp.inf, vals)
        if k == 0:
            top_max = m
        idx_cols.append(idx)

    unnorm = jnp.where(selected, jnp.exp(logits - top_max), 0.0)
    denom = jnp.sum(unnorm, axis=1, keepdims=True)
    scores_ref[...] = unnorm / denom
    idx_ref[...] = jnp.concatenate(idx_cols, axis=1)


@functools.partial(jax.jit, static_argnames=())
def kernel(hidden_states, W, b):
    B, S, H = hidden_states.shape
    T = B * S
    x = hidden_states.reshape(T, H)
    wt = W.T
    b2 = b.reshape(1, NUM_EXPERTS)

    grid = (T // TOKEN_BLOCK,)
    scores, indices = pl.pallas_call(
        _router_block,
        grid=grid,
        in_specs=[
            pl.BlockSpec((TOKEN_BLOCK, H), lambda i: (i, 0)),
            pl.BlockSpec((H, NUM_EXPERTS), lambda i: (0, 0)),
            pl.BlockSpec((1, NUM_EXPERTS), lambda i: (0, 0)),
        ],
        out_specs=[
            pl.BlockSpec((TOKEN_BLOCK, NUM_EXPERTS), lambda i: (i, 0)),
            pl.BlockSpec((TOKEN_BLOCK, TOP_K), lambda i: (i, 0)),
        ],
        out_shape=[
            jax.ShapeDtypeStruct((T, NUM_EXPERTS), jnp.float32),
            jax.ShapeDtypeStruct((T, TOP_K), jnp.int32),
        ],
        compiler_params=pltpu.CompilerParams(
            dimension_semantics=("arbitrary",),
        ),
    )(x, wt, b2)
    return scores, indices


# speedup vs baseline: 5.3863x; 1.1147x over previous
import functools

import jax
import jax.numpy as jnp
from jax.experimental import pallas as pl
from jax.experimental.pallas import tpu as pltpu

TOP_K = 8
NUM_EXPERTS = 64
HIDDEN = 4096
TOKEN_BLOCK = 512


def _router_block(x_ref, wt_ref, b_ref, scores_ref, idx_ref):
    x = x_ref[...]
    logits = jnp.dot(x, wt_ref[...], preferred_element_type=jnp.float32)
    logits = logits + b_ref[...]

    tb = logits.shape[0]
    e_iota = jax.lax.broadcasted_iota(
        jnp.int32, (tb, NUM_EXPERTS), 1).astype(jnp.float32)

    vals = logits
    top_max = None
    idx_cols = []
    for k in range(TOP_K):
        m = jnp.max(vals, axis=1, keepdims=True)
        hit = vals == m
        idx = jnp.min(jnp.where(hit, e_iota, float(NUM_EXPERTS)), axis=1,
                      keepdims=True)
        vals = jnp.where(e_iota == idx, -jnp.inf, vals)
        if k == 0:
            top_max = m
        idx_cols.append(idx)

    selected = vals == -jnp.inf
    unnorm = jnp.where(selected, jnp.exp(logits - top_max), 0.0)
    denom = jnp.sum(unnorm, axis=1, keepdims=True)
    scores_ref[...] = unnorm / denom
    idx_ref[...] = jnp.concatenate(idx_cols, axis=1).astype(jnp.int32)


@functools.partial(jax.jit, static_argnames=())
def kernel(hidden_states, W, b):
    B, S, H = hidden_states.shape
    T = B * S
    x = hidden_states.reshape(T, H)
    wt = W.T
    b2 = b.reshape(1, NUM_EXPERTS)

    grid = (T // TOKEN_BLOCK,)
    scores, indices = pl.pallas_call(
        _router_block,
        grid=grid,
        in_specs=[
            pl.BlockSpec((TOKEN_BLOCK, H), lambda i: (i, 0)),
            pl.BlockSpec((H, NUM_EXPERTS), lambda i: (0, 0)),
            pl.BlockSpec((1, NUM_EXPERTS), lambda i: (0, 0)),
        ],
        out_specs=[
            pl.BlockSpec((TOKEN_BLOCK, NUM_EXPERTS), lambda i: (i, 0)),
            pl.BlockSpec((TOKEN_BLOCK, TOP_K), lambda i: (i, 0)),
        ],
        out_shape=[
            jax.ShapeDtypeStruct((T, NUM_EXPERTS), jnp.float32),
            jax.ShapeDtypeStruct((T, TOP_K), jnp.int32),
        ],
        compiler_params=pltpu.CompilerParams(
            dimension_semantics=("arbitrary",),
        ),
    )(x, wt, b2)
    return scores, indices


# speedup vs baseline: 5.9458x; 1.1039x over previous
import functools

import jax
import jax.numpy as jnp
from jax.experimental import pallas as pl
from jax.experimental.pallas import tpu as pltpu

TOP_K = 8
NUM_EXPERTS = 64
HIDDEN = 4096
TOKEN_BLOCK = 1024


def _router_block(x_ref, wt_ref, b_ref, scores_ref, idx_ref):
    x = x_ref[...]
    logits = jnp.dot(x, wt_ref[...], preferred_element_type=jnp.float32)
    logits = logits + b_ref[...]

    tb = logits.shape[0]
    e_iota = jax.lax.broadcasted_iota(
        jnp.int32, (tb, NUM_EXPERTS), 1).astype(jnp.float32)

    vals = logits
    top_max = None
    idx_cols = []
    for k in range(TOP_K):
        m = jnp.max(vals, axis=1, keepdims=True)
        hit = vals == m
        idx = jnp.min(jnp.where(hit, e_iota, float(NUM_EXPERTS)), axis=1,
                      keepdims=True)
        vals = jnp.where(e_iota == idx, -jnp.inf, vals)
        if k == 0:
            top_max = m
        idx_cols.append(idx)

    selected = vals == -jnp.inf
    unnorm = jnp.where(selected, jnp.exp(logits - top_max), 0.0)
    denom = jnp.sum(unnorm, axis=1, keepdims=True)
    scores_ref[...] = unnorm / denom
    idx_ref[...] = jnp.concatenate(idx_cols, axis=1).astype(jnp.int32)


@functools.partial(jax.jit, static_argnames=())
def kernel(hidden_states, W, b):
    B, S, H = hidden_states.shape
    T = B * S
    x = hidden_states.reshape(T, H)
    wt = W.T
    b2 = b.reshape(1, NUM_EXPERTS)

    grid = (T // TOKEN_BLOCK,)
    scores, indices = pl.pallas_call(
        _router_block,
        grid=grid,
        in_specs=[
            pl.BlockSpec((TOKEN_BLOCK, H), lambda i: (i, 0)),
            pl.BlockSpec((H, NUM_EXPERTS), lambda i: (0, 0)),
            pl.BlockSpec((1, NUM_EXPERTS), lambda i: (0, 0)),
        ],
        out_specs=[
            pl.BlockSpec((TOKEN_BLOCK, NUM_EXPERTS), lambda i: (i, 0)),
            pl.BlockSpec((TOKEN_BLOCK, TOP_K), lambda i: (i, 0)),
        ],
        out_shape=[
            jax.ShapeDtypeStruct((T, NUM_EXPERTS), jnp.float32),
            jax.ShapeDtypeStruct((T, TOP_K), jnp.int32),
        ],
        compiler_params=pltpu.CompilerParams(
            dimension_semantics=("arbitrary",),
        ),
    )(x, wt, b2)
    return scores, indices
